# Initial kernel scaffold; baseline (speedup 1.0000x reference)
import jax
import jax.numpy as jnp
from jax.experimental import pallas as pl


def kernel(x, pos, edge_index, batch, W_src, W_dst, W_lin, Wp1, bp1, gp1, bbp1, Wp2, bp2, Wa1, ba1, ga1, bba1, Wa2, ba2, Wd, bd, gd, bbd):
    raise NotImplementedError("write your pallas kernel here")


# speedup vs baseline: 1.0181x; 1.0181x over previous
import functools
import jax
import jax.numpy as jnp
from jax.experimental import pallas as pl
from jax.experimental.pallas import tpu as pltpu

_EPS_BN = 1e-5


def _nodeproj_body(x_ref, posp_ref, ws_ref, wd_ref, wl_ref, wp1_ref,
                   asrc_ref, adst_ref, v_ref, p_ref):
    x = x_ref[...]
    asrc_ref[...] = jnp.dot(x, ws_ref[...], preferred_element_type=jnp.float32)
    adst_ref[...] = jnp.dot(x, wd_ref[...], preferred_element_type=jnp.float32)
    v_ref[...] = jnp.dot(x, wl_ref[...], preferred_element_type=jnp.float32)
    p_ref[...] = jnp.dot(posp_ref[...], wp1_ref[...],
                         preferred_element_type=jnp.float32)


def _stats_body(g_ref, o_ref):
    @pl.when(pl.program_id(0) == 0)
    def _():
        o_ref[...] = jnp.zeros_like(o_ref)
    g = g_ref[...]
    s = jnp.sum(g, axis=0, keepdims=True)
    ss = jnp.sum(g * g, axis=0, keepdims=True)
    o_ref[...] += jnp.concatenate([s, ss, jnp.zeros((6, g.shape[1]), g.dtype)], axis=0)


def _edge2_body(g1_ref, g2_ref, st1_ref, gp1_ref, bbp1_ref, wp2_ref, bp2_ref,
                wa1_ref, delta_ref, h2_ref, st2_ref, *, n_rows):
    @pl.when(pl.program_id(0) == 0)
    def _():
        st2_ref[...] = jnp.zeros_like(st2_ref)
    inv_n = 1.0 / n_rows
    mu = st1_ref[0:1, :] * inv_n
    var = st1_ref[1:2, :] * inv_n - mu * mu
    h1 = g1_ref[...]
    hn = gp1_ref[...] * (h1 - mu) * jax.lax.rsqrt(var + _EPS_BN) + bbp1_ref[...]
    hn = jnp.maximum(hn, 0.0)
    delta = jnp.dot(hn, wp2_ref[...], preferred_element_type=jnp.float32) + bp2_ref[...]
    delta_ref[...] = delta
    h2 = jnp.dot(g2_ref[...] + delta, wa1_ref[...], preferred_element_type=jnp.float32)
    h2_ref[...] = h2
    s = jnp.sum(h2, axis=0, keepdims=True)
    ss = jnp.sum(h2 * h2, axis=0, keepdims=True)
    st2_ref[...] += jnp.concatenate([s, ss, jnp.zeros((6, h2.shape[1]), h2.dtype)], axis=0)


def _edge3_body(h2_ref, delta_ref, g3_ref, st2_ref, ga1_ref, bba1_ref,
                wa2_ref, ba2_ref, ex_ref, y_ref, *, n_rows):
    inv_n = 1.0 / n_rows
    mu = st2_ref[0:1, :] * inv_n
    var = st2_ref[1:2, :] * inv_n - mu * mu
    hn = ga1_ref[...] * (h2_ref[...] - mu) * jax.lax.rsqrt(var + _EPS_BN) + bba1_ref[...]
    hn = jnp.maximum(hn, 0.0)
    alpha = jnp.dot(hn, wa2_ref[...], preferred_element_type=jnp.float32) + ba2_ref[...]
    ex = jnp.exp(alpha)
    ex_ref[...] = ex
    y_ref[...] = ex * (g3_ref[...] + delta_ref[...])


def _down1_body(num_ref, s_ref, wdn_ref, bdn_ref, t_ref, st_ref):
    @pl.when(pl.program_id(0) == 0)
    def _():
        st_ref[...] = jnp.zeros_like(st_ref)
    out = num_ref[...] / (s_ref[...] + 1e-16)
    t = jnp.dot(out, wdn_ref[...], preferred_element_type=jnp.float32) + bdn_ref[...]
    t_ref[...] = t
    s = jnp.sum(t, axis=0, keepdims=True)
    ss = jnp.sum(t * t, axis=0, keepdims=True)
    st_ref[...] += jnp.concatenate([s, ss, jnp.zeros((6, t.shape[1]), t.dtype)], axis=0)


def _down2_body(t_ref, st_ref, gd_ref, bbd_ref, h_ref, *, n_rows):
    inv_n = 1.0 / n_rows
    mu = st_ref[0:1, :] * inv_n
    var = st_ref[1:2, :] * inv_n - mu * mu
    hn = gd_ref[...] * (t_ref[...] - mu) * jax.lax.rsqrt(var + _EPS_BN) + bbd_ref[...]
    h_ref[...] = jnp.maximum(hn, 0.0)


def _row_spec(bm, d):
    return pl.BlockSpec((bm, d), lambda i: (i, 0))


def _full_spec(shape):
    return pl.BlockSpec(shape, lambda i: tuple(0 for _ in shape))


def _fps(pos, num_samples):
    pos = jax.lax.stop_gradient(pos)
    dist0 = jnp.sum((pos - pos[0]) ** 2, axis=1)
    idxs0 = jnp.zeros((num_samples,), dtype=jnp.int32)

    def body(i, state):
        dist, idxs = state
        nxt = jnp.argmax(dist).astype(jnp.int32)
        idxs = idxs.at[i].set(nxt)
        d = jnp.sum((pos - pos[nxt]) ** 2, axis=1)
        return jnp.minimum(dist, d), idxs

    _, idxs = jax.lax.fori_loop(1, num_samples, body, (dist0, idxs0))
    return jnp.sort(idxs)


@jax.jit
def kernel(x, pos, edge_index, batch, W_src, W_dst, W_lin, Wp1, bp1, gp1, bbp1,
           Wp2, bp2, Wa1, ba1, ga1, bba1, Wa2, ba2, Wd, bd, gd, bbd):
    N, D = x.shape
    E = edge_index.shape[1]
    M = N // 2
    src = edge_index[0]
    dst = edge_index[1]

    f32 = jnp.float32
    row1 = lambda v: v.reshape(1, D)

    BN_BLK = 2000
    posp = jnp.concatenate([pos, jnp.zeros((N, 5), f32)], axis=1)
    wp1p = jnp.concatenate([Wp1, jnp.zeros((5, D), f32)], axis=0)
    grid_n = N // BN_BLK
    a_src, a_dst, v, P = pl.pallas_call(
        _nodeproj_body,
        grid=(grid_n,),
        in_specs=[_row_spec(BN_BLK, D), _row_spec(BN_BLK, 8),
                  _full_spec((D, D)), _full_spec((D, D)), _full_spec((D, D)),
                  _full_spec((8, D))],
        out_specs=[_row_spec(BN_BLK, D)] * 4,
        out_shape=[jax.ShapeDtypeStruct((N, D), f32)] * 4,
    )(x, posp, W_src, W_dst, W_lin, wp1p)

    g1 = P[dst] - P[src]
    g2 = a_dst[dst] - a_src[src]
    g3 = v[src]

    BE = 2000
    grid_e = E // BE

    st1 = pl.pallas_call(
        _stats_body,
        grid=(grid_e,),
        in_specs=[_row_spec(BE, D)],
        out_specs=pl.BlockSpec((8, D), lambda i: (0, 0)),
        out_shape=jax.ShapeDtypeStruct((8, D), f32),
    )(g1)

    delta, h2, st2 = pl.pallas_call(
        functools.partial(_edge2_body, n_rows=float(E)),
        grid=(grid_e,),
        in_specs=[_row_spec(BE, D), _row_spec(BE, D), _full_spec((8, D)),
                  _full_spec((1, D)), _full_spec((1, D)), _full_spec((D, D)),
                  _full_spec((1, D)), _full_spec((D, D))],
        out_specs=[_row_spec(BE, D), _row_spec(BE, D),
                   pl.BlockSpec((8, D), lambda i: (0, 0))],
        out_shape=[jax.ShapeDtypeStruct((E, D), f32),
                   jax.ShapeDtypeStruct((E, D), f32),
                   jax.ShapeDtypeStruct((8, D), f32)],
    )(g1, g2, st1, row1(gp1), row1(bbp1), Wp2, row1(bp2), Wa1)

    ex, y = pl.pallas_call(
        functools.partial(_edge3_body, n_rows=float(E)),
        grid=(grid_e,),
        in_specs=[_row_spec(BE, D), _row_spec(BE, D), _row_spec(BE, D),
                  _full_spec((8, D)), _full_spec((1, D)), _full_spec((1, D)),
                  _full_spec((D, D)), _full_spec((1, D))],
        out_specs=[_row_spec(BE, D), _row_spec(BE, D)],
        out_shape=[jax.ShapeDtypeStruct((E, D), f32),
                   jax.ShapeDtypeStruct((E, D), f32)],
    )(h2, delta, g3, st2, row1(ga1), row1(bba1), Wa2, row1(ba2))

    s = jax.ops.segment_sum(ex, dst, num_segments=N)
    num = jax.ops.segment_sum(y, dst, num_segments=N)

    t, stn = pl.pallas_call(
        _down1_body,
        grid=(grid_n,),
        in_specs=[_row_spec(BN_BLK, D), _row_spec(BN_BLK, D),
                  _full_spec((D, D)), _full_spec((1, D))],
        out_specs=[_row_spec(BN_BLK, D), pl.BlockSpec((8, D), lambda i: (0, 0))],
        out_shape=[jax.ShapeDtypeStruct((N, D), f32),
                   jax.ShapeDtypeStruct((8, D), f32)],
    )(num, s, Wd, row1(bd))

    h = pl.pallas_call(
        functools.partial(_down2_body, n_rows=float(N)),
        grid=(grid_n,),
        in_specs=[_row_spec(BN_BLK, D), _full_spec((8, D)),
                  _full_spec((1, D)), _full_spec((1, D))],
        out_specs=_row_spec(BN_BLK, D),
        out_shape=jax.ShapeDtypeStruct((N, D), f32),
    )(t, stn, row1(gd), row1(bbd))

    pooled = jnp.maximum(jax.ops.segment_max(h[src], dst, num_segments=N), h)
    idx = _fps(pos, M)
    return pooled[idx], pos[idx]
